# baseline (speedup 1.0000x reference)
import functools

import jax
import jax.numpy as jnp
from jax import lax
from jax.experimental import pallas as pl
from jax.experimental.pallas import tpu as pltpu
from jax.experimental.pallas import tpu_sc as plsc

D_T = 32
D_COND = 48
N_POS = 3
K = 20
D_H = 32
B = 2
N = 4096
D_Z = 54

TW = 128
RB = 256
RM = 128
RBK = 512
E = B * N * K

SC_NC = 2
SC_NS = 16
NW = SC_NC * SC_NS
EW = E // NW
GC = 512
GS = GC // 128
NG = EW // GC


def _cond_kernel(t_ref, c_ref, W1_ref, b1_ref, W2_ref, b2_ref, W3_ref, b3_ref, out_ref):
    t = t_ref[...]
    half = D_T // 2
    i = lax.broadcasted_iota(jnp.int32, (1, half), 1).astype(jnp.float32)
    freqs = jnp.exp(-jnp.log(10000.0) * i / (half - 1))
    args = t * freqs
    emb = jnp.concatenate([jnp.sin(args), jnp.cos(args)], axis=-1)
    x = jnp.concatenate([emb, c_ref[...]], axis=-1)
    h = jax.nn.gelu(x @ W1_ref[...] + b1_ref[...])
    h = jax.nn.gelu(h @ W2_ref[...] + b2_ref[...])
    out_ref[...] = h @ W3_ref[...] + b3_ref[...]


def _table_kernel(z_ref, cond_ref, Wq_ref, Wk_ref, Wv_ref, Wg_ref, bg_ref,
                  q_ref, T_ref):
    z = z_ref[0]
    s = z[:, 2 * N_POS:] + cond_ref[0]
    q_ref[0] = s @ Wq_ref[...]
    kt = s @ Wk_ref[...]
    vt = s @ Wv_ref[...]
    gt = s @ Wg_ref[...] + bg_ref[...]
    pos = z[:, 0:N_POS]
    vel = z[:, N_POS:2 * N_POS]
    pad = jnp.zeros((z.shape[0], TW - 89), jnp.float32)
    T_ref[...] = jnp.concatenate([kt, vt, gt, pos, vel, pad], axis=-1)


def _knn_kernel(z_ref, posT_ref, tgt_ref):
    b = pl.program_id(0)
    pr = z_ref[0]
    rows = pr.shape[0]
    d2 = None
    for c in range(N_POS):
        diff = posT_ref[0, c:c + 1, :] - pr[:, c:c + 1]
        d2 = diff * diff if d2 is None else d2 + diff * diff
    iota = lax.broadcasted_iota(jnp.int32, (rows, N), 1)
    bits = lax.bitcast_convert_type(d2, jnp.int32)
    key = lax.bitcast_convert_type(
        ((bits & jnp.int32(-4096)) | iota) + jnp.int32(1 << 23), jnp.float32)
    cols = []
    base = b * N
    for _ in range(K):
        m = jnp.min(key, axis=1, keepdims=True)
        mi = lax.bitcast_convert_type(m, jnp.int32)
        cols.append((mi & jnp.int32(4095)) + base)
        key = jnp.where(key == m, jnp.float32(jnp.inf), key)
    tgt_ref[0] = jnp.concatenate(cols, axis=-1)


def _mp_kernel(z_ref, q_ref, g_ref, Wo_ref, bo_ref, gamma_ref, out_ref):
    zb = z_ref[0]
    q = q_ref[0]
    G = g_ref[...].reshape(RM, K, TW)
    kt = G[:, :, 0:32]
    vt = G[:, :, 32:80]
    gt = G[:, :, 80:83]
    post = G[:, :, 83:86]
    velt = G[:, :, 86:89]
    pos = zb[:, 0:N_POS]
    rel = post - pos[:, None, :]
    dist2 = jnp.sum(rel * rel, axis=-1)
    gamma = gamma_ref[0, 0]
    logits = jnp.sum(q[:, None, :] * kt, axis=-1) / jnp.sqrt(jnp.float32(D_H)) - gamma * dist2
    att = jax.nn.softmax(logits, axis=-1)
    ns = jnp.sum(att[:, :, None] * vt, axis=1)
    new_s = ns @ Wo_ref[...] + bo_ref[...]
    aw = att[:, :, None]
    dpos = jnp.sum(aw * gt[:, :, 0:1] * rel, axis=1)
    dv = jnp.sum(aw * (gt[:, :, 1:2] * rel + gt[:, :, 2:3] * velt), axis=1)
    h = jnp.concatenate([dpos, dv, new_s], axis=-1)
    out_ref[0] = zb + h


def _gather_rows(T, idx2):
    mesh = plsc.VectorSubcoreMesh(core_axis_name="c", subcore_axis_name="s")

    @functools.partial(
        pl.kernel,
        out_type=jax.ShapeDtypeStruct((E, TW), jnp.float32),
        mesh=mesh,
        scratch_types=[
            pltpu.VMEM((GS, 128), jnp.int32),
            pltpu.VMEM((GC, TW), jnp.float32),
            pltpu.SemaphoreType.DMA,
        ],
    )
    def gather_k(T_hbm, idx_hbm, out_hbm, idx_v, rows_v, sem):
        cid = lax.axis_index("c")
        sid = lax.axis_index("s")
        wid = sid * SC_NC + cid

        def body(g, carry):
            base = wid * EW + g * GC
            row0 = wid * (EW // 128) + g * GS
            pltpu.sync_copy(idx_hbm.at[pl.ds(row0, GS)], idx_v)
            for r in range(GS):
                pltpu.async_copy(
                    T_hbm.at[idx_v.at[r]], rows_v.at[pl.ds(r * 128, 128)], sem
                ).wait()
            pltpu.sync_copy(rows_v, out_hbm.at[pl.ds(base, GC)])
            return carry

        lax.fori_loop(0, NG, body, 0)

    return gather_k(T, idx2)


def kernel(z, t, conditioning, mask, W1, b1, W2, b2, W3, b3, Wq, Wk, Wv, Wo, bo, Wg, bg, gamma):
    del mask
    f32 = jnp.float32

    cond = pl.pallas_call(
        _cond_kernel,
        out_shape=jax.ShapeDtypeStruct((B, D_COND), f32),
    )(t.reshape(B, 1).astype(f32), conditioning,
      W1, b1.reshape(1, -1), W2, b2.reshape(1, -1), W3, b3.reshape(1, -1))
    cond3 = cond.reshape(B, 1, D_COND)

    nbt = N // RB
    q, T = pl.pallas_call(
        _table_kernel,
        grid=(B, nbt),
        in_specs=[
            pl.BlockSpec((1, RB, D_Z), lambda b_, i: (b_, i, 0)),
            pl.BlockSpec((1, 1, D_COND), lambda b_, i: (b_, 0, 0)),
            pl.BlockSpec((D_COND, D_H), lambda b_, i: (0, 0)),
            pl.BlockSpec((D_COND, D_H), lambda b_, i: (0, 0)),
            pl.BlockSpec((D_COND, D_COND), lambda b_, i: (0, 0)),
            pl.BlockSpec((D_COND, 3), lambda b_, i: (0, 0)),
            pl.BlockSpec((1, 3), lambda b_, i: (0, 0)),
        ],
        out_specs=[
            pl.BlockSpec((1, RB, D_H), lambda b_, i: (b_, i, 0)),
            pl.BlockSpec((RB, TW), lambda b_, i: (b_ * nbt + i, 0)),
        ],
        out_shape=[
            jax.ShapeDtypeStruct((B, N, D_H), f32),
            jax.ShapeDtypeStruct((B * N, TW), f32),
        ],
    )(z, cond3, Wq, Wk, Wv, Wg, bg.reshape(1, 3))

    posT = jnp.transpose(z[..., :N_POS], (0, 2, 1))
    tgt = pl.pallas_call(
        _knn_kernel,
        grid=(B, N // RBK),
        in_specs=[
            pl.BlockSpec((1, RBK, D_Z), lambda b_, i: (b_, i, 0)),
            pl.BlockSpec((1, N_POS, N), lambda b_, i: (b_, 0, 0)),
        ],
        out_specs=pl.BlockSpec((1, RBK, K), lambda b_, i: (b_, i, 0)),
        out_shape=jax.ShapeDtypeStruct((B, N, K), jnp.int32),
    )(z, posT)

    gath = _gather_rows(T, tgt.reshape(E // 128, 128))

    nbm = N // RM
    out = pl.pallas_call(
        _mp_kernel,
        grid=(B, nbm),
        in_specs=[
            pl.BlockSpec((1, RM, D_Z), lambda b_, i: (b_, i, 0)),
            pl.BlockSpec((1, RM, D_H), lambda b_, i: (b_, i, 0)),
            pl.BlockSpec((RM * K, TW), lambda b_, i: (b_ * nbm + i, 0)),
            pl.BlockSpec((D_COND, D_COND), lambda b_, i: (0, 0)),
            pl.BlockSpec((1, D_COND), lambda b_, i: (0, 0)),
            pl.BlockSpec((1, 1), lambda b_, i: (0, 0)),
        ],
        out_specs=pl.BlockSpec((1, RM, D_Z), lambda b_, i: (b_, i, 0)),
        out_shape=jax.ShapeDtypeStruct((B, N, D_Z), f32),
    )(z, q, gath, Wo, bo.reshape(1, -1), gamma.reshape(1, 1))

    return out


# speedup vs baseline: 1.0228x; 1.0228x over previous
import functools

import jax
import jax.numpy as jnp
from jax import lax
from jax.experimental import pallas as pl
from jax.experimental.pallas import tpu as pltpu
from jax.experimental.pallas import tpu_sc as plsc

D_T = 32
D_COND = 48
N_POS = 3
K = 20
D_H = 32
B = 2
N = 4096
D_Z = 54

TW = 128
RB = 256
RM = 256
RBK = 256
E = B * N * K

SC_NC = 2
SC_NS = 16
NW = SC_NC * SC_NS
EW = E // NW
GC = 512
GS = GC // 128
NG = EW // GC


def _cond_kernel(t_ref, c_ref, W1_ref, b1_ref, W2_ref, b2_ref, W3_ref, b3_ref, out_ref):
    t = t_ref[...]
    half = D_T // 2
    i = lax.broadcasted_iota(jnp.int32, (1, half), 1).astype(jnp.float32)
    freqs = jnp.exp(-jnp.log(10000.0) * i / (half - 1))
    args = t * freqs
    emb = jnp.concatenate([jnp.sin(args), jnp.cos(args)], axis=-1)
    x = jnp.concatenate([emb, c_ref[...]], axis=-1)
    h = jax.nn.gelu(x @ W1_ref[...] + b1_ref[...])
    h = jax.nn.gelu(h @ W2_ref[...] + b2_ref[...])
    out_ref[...] = h @ W3_ref[...] + b3_ref[...]


def _table_kernel(z_ref, cond_ref, Wq_ref, Wk_ref, Wv_ref, Wg_ref, bg_ref,
                  q_ref, T_ref):
    z = z_ref[0]
    s = z[:, 2 * N_POS:] + cond_ref[0]
    q_ref[0] = s @ Wq_ref[...]
    kt = s @ Wk_ref[...]
    vt = s @ Wv_ref[...]
    gt = s @ Wg_ref[...] + bg_ref[...]
    pos = z[:, 0:N_POS]
    vel = z[:, N_POS:2 * N_POS]
    pad = jnp.zeros((z.shape[0], TW - 89), jnp.float32)
    T_ref[...] = jnp.concatenate([kt, vt, gt, pos, vel, pad], axis=-1)


def _knn_kernel(z_ref, posT_ref, tgt_ref):
    b = pl.program_id(0)
    pr = z_ref[0]
    rows = pr.shape[0]
    d2 = None
    for c in range(N_POS):
        diff = posT_ref[0, c:c + 1, :] - pr[:, c:c + 1]
        d2 = diff * diff if d2 is None else d2 + diff * diff
    iota = lax.broadcasted_iota(jnp.int32, (rows, N), 1)
    bits = lax.bitcast_convert_type(d2, jnp.int32)
    key = lax.bitcast_convert_type(
        ((bits & jnp.int32(-4096)) | iota) + jnp.int32(1 << 23), jnp.float32)
    cols = []
    base = b * N
    for _ in range(K):
        m = jnp.min(key, axis=1, keepdims=True)
        mi = lax.bitcast_convert_type(m, jnp.int32)
        cols.append((mi & jnp.int32(4095)) + base)
        key = jnp.where(key == m, jnp.float32(jnp.inf), key)
    tgt_ref[0] = jnp.concatenate(cols, axis=-1)


def _mp_kernel(z_ref, q_ref, g_ref, Wo_ref, bo_ref, gamma_ref, out_ref):
    zb = z_ref[0]
    q = q_ref[0]
    G = g_ref[...].reshape(RM, K, TW)
    kt = G[:, :, 0:32]
    vt = G[:, :, 32:80]
    gt = G[:, :, 80:83]
    post = G[:, :, 83:86]
    velt = G[:, :, 86:89]
    pos = zb[:, 0:N_POS]
    rel = post - pos[:, None, :]
    dist2 = jnp.sum(rel * rel, axis=-1)
    gamma = gamma_ref[0, 0]
    logits = jnp.sum(q[:, None, :] * kt, axis=-1) / jnp.sqrt(jnp.float32(D_H)) - gamma * dist2
    att = jax.nn.softmax(logits, axis=-1)
    ns = jnp.sum(att[:, :, None] * vt, axis=1)
    new_s = ns @ Wo_ref[...] + bo_ref[...]
    aw = att[:, :, None]
    dpos = jnp.sum(aw * gt[:, :, 0:1] * rel, axis=1)
    dv = jnp.sum(aw * (gt[:, :, 1:2] * rel + gt[:, :, 2:3] * velt), axis=1)
    h = jnp.concatenate([dpos, dv, new_s], axis=-1)
    out_ref[0] = zb + h


def _gather_rows(T, idx2):
    mesh = plsc.VectorSubcoreMesh(core_axis_name="c", subcore_axis_name="s")

    @functools.partial(
        pl.kernel,
        out_type=jax.ShapeDtypeStruct((E, TW), jnp.float32),
        mesh=mesh,
        scratch_types=[
            pltpu.VMEM((GS, 128), jnp.int32),
            pltpu.VMEM((GC, TW), jnp.float32),
            pltpu.SemaphoreType.DMA,
        ],
    )
    def gather_k(T_hbm, idx_hbm, out_hbm, idx_v, rows_v, sem):
        cid = lax.axis_index("c")
        sid = lax.axis_index("s")
        wid = sid * SC_NC + cid

        def body(g, carry):
            base = wid * EW + g * GC
            row0 = wid * (EW // 128) + g * GS
            pltpu.sync_copy(idx_hbm.at[pl.ds(row0, GS)], idx_v)
            for r in range(GS):
                pltpu.async_copy(
                    T_hbm.at[idx_v.at[r]], rows_v.at[pl.ds(r * 128, 128)], sem
                ).wait()
            pltpu.sync_copy(rows_v, out_hbm.at[pl.ds(base, GC)])
            return carry

        lax.fori_loop(0, NG, body, 0)

    return gather_k(T, idx2)


def kernel(z, t, conditioning, mask, W1, b1, W2, b2, W3, b3, Wq, Wk, Wv, Wo, bo, Wg, bg, gamma):
    del mask
    f32 = jnp.float32

    cond = pl.pallas_call(
        _cond_kernel,
        out_shape=jax.ShapeDtypeStruct((B, D_COND), f32),
    )(t.reshape(B, 1).astype(f32), conditioning,
      W1, b1.reshape(1, -1), W2, b2.reshape(1, -1), W3, b3.reshape(1, -1))
    cond3 = cond.reshape(B, 1, D_COND)

    nbt = N // RB
    q, T = pl.pallas_call(
        _table_kernel,
        grid=(B, nbt),
        in_specs=[
            pl.BlockSpec((1, RB, D_Z), lambda b_, i: (b_, i, 0)),
            pl.BlockSpec((1, 1, D_COND), lambda b_, i: (b_, 0, 0)),
            pl.BlockSpec((D_COND, D_H), lambda b_, i: (0, 0)),
            pl.BlockSpec((D_COND, D_H), lambda b_, i: (0, 0)),
            pl.BlockSpec((D_COND, D_COND), lambda b_, i: (0, 0)),
            pl.BlockSpec((D_COND, 3), lambda b_, i: (0, 0)),
            pl.BlockSpec((1, 3), lambda b_, i: (0, 0)),
        ],
        out_specs=[
            pl.BlockSpec((1, RB, D_H), lambda b_, i: (b_, i, 0)),
            pl.BlockSpec((RB, TW), lambda b_, i: (b_ * nbt + i, 0)),
        ],
        out_shape=[
            jax.ShapeDtypeStruct((B, N, D_H), f32),
            jax.ShapeDtypeStruct((B * N, TW), f32),
        ],
    )(z, cond3, Wq, Wk, Wv, Wg, bg.reshape(1, 3))

    posT = jnp.transpose(z[..., :N_POS], (0, 2, 1))
    tgt = pl.pallas_call(
        _knn_kernel,
        grid=(B, N // RBK),
        in_specs=[
            pl.BlockSpec((1, RBK, D_Z), lambda b_, i: (b_, i, 0)),
            pl.BlockSpec((1, N_POS, N), lambda b_, i: (b_, 0, 0)),
        ],
        out_specs=pl.BlockSpec((1, RBK, K), lambda b_, i: (b_, i, 0)),
        out_shape=jax.ShapeDtypeStruct((B, N, K), jnp.int32),
    )(z, posT)

    gath = _gather_rows(T, tgt.reshape(E // 128, 128))

    nbm = N // RM
    out = pl.pallas_call(
        _mp_kernel,
        grid=(B, nbm),
        in_specs=[
            pl.BlockSpec((1, RM, D_Z), lambda b_, i: (b_, i, 0)),
            pl.BlockSpec((1, RM, D_H), lambda b_, i: (b_, i, 0)),
            pl.BlockSpec((RM * K, TW), lambda b_, i: (b_ * nbm + i, 0)),
            pl.BlockSpec((D_COND, D_COND), lambda b_, i: (0, 0)),
            pl.BlockSpec((1, D_COND), lambda b_, i: (0, 0)),
            pl.BlockSpec((1, 1), lambda b_, i: (0, 0)),
        ],
        out_specs=pl.BlockSpec((1, RM, D_Z), lambda b_, i: (b_, i, 0)),
        out_shape=jax.ShapeDtypeStruct((B, N, D_Z), f32),
    )(z, q, gath, Wo, bo.reshape(1, -1), gamma.reshape(1, 1))

    return out


# speedup vs baseline: 1.0508x; 1.0274x over previous
import functools

import jax
import jax.numpy as jnp
from jax import lax
from jax.experimental import pallas as pl
from jax.experimental.pallas import tpu as pltpu
from jax.experimental.pallas import tpu_sc as plsc

D_T = 32
D_COND = 48
N_POS = 3
K = 20
D_H = 32
B = 2
N = 4096
D_Z = 54

TW = 128
RB = 256
RM = 256
RBK = 256
E = B * N * K

SC_NC = 2
SC_NS = 16
NW = SC_NC * SC_NS
EW = E // NW
GC = 512
GS = GC // 128
NG = EW // GC


def _cond_kernel(t_ref, c_ref, W1_ref, b1_ref, W2_ref, b2_ref, W3_ref, b3_ref, out_ref):
    t = t_ref[...]
    half = D_T // 2
    i = lax.broadcasted_iota(jnp.int32, (1, half), 1).astype(jnp.float32)
    freqs = jnp.exp(-jnp.log(10000.0) * i / (half - 1))
    args = t * freqs
    emb = jnp.concatenate([jnp.sin(args), jnp.cos(args)], axis=-1)
    x = jnp.concatenate([emb, c_ref[...]], axis=-1)
    h = jax.nn.gelu(x @ W1_ref[...] + b1_ref[...])
    h = jax.nn.gelu(h @ W2_ref[...] + b2_ref[...])
    out_ref[...] = h @ W3_ref[...] + b3_ref[...]


def _table_kernel(z_ref, cond_ref, Wq_ref, Wk_ref, Wv_ref, Wg_ref, bg_ref,
                  q_ref, T_ref):
    z = z_ref[0]
    s = z[:, 2 * N_POS:] + cond_ref[0]
    q_ref[0] = s @ Wq_ref[...]
    kt = s @ Wk_ref[...]
    vt = s @ Wv_ref[...]
    gt = s @ Wg_ref[...] + bg_ref[...]
    pos = z[:, 0:N_POS]
    vel = z[:, N_POS:2 * N_POS]
    pad = jnp.zeros((z.shape[0], TW - 89), jnp.float32)
    T_ref[...] = jnp.concatenate([kt, vt, gt, pos, vel, pad], axis=-1)


def _knn_kernel(z_ref, posT_ref, tgt_ref):
    b = pl.program_id(0)
    pr = z_ref[0]
    rows = pr.shape[0]
    d2 = None
    for c in range(N_POS):
        diff = posT_ref[0, c:c + 1, :] - pr[:, c:c + 1]
        d2 = diff * diff if d2 is None else d2 + diff * diff
    iota = lax.broadcasted_iota(jnp.int32, (rows, N), 1)
    bits = lax.bitcast_convert_type(d2, jnp.int32)
    key = lax.bitcast_convert_type(
        ((bits & jnp.int32(-4096)) | iota) + jnp.int32(1 << 23), jnp.float32)
    cols = []
    base = b * N
    for _ in range(K):
        m = jnp.min(key, axis=1, keepdims=True)
        mi = lax.bitcast_convert_type(m, jnp.int32)
        cols.append((mi & jnp.int32(4095)) + base)
        key = jnp.where(key == m, jnp.float32(jnp.inf), key)
    tgt_ref[0] = jnp.concatenate(cols, axis=-1)


def _mp_kernel(z_ref, q_ref, g_ref, Wo_ref, bo_ref, gamma_ref, out_ref):
    zb = z_ref[0]
    q = q_ref[0]
    G = g_ref[...].reshape(RM, K, TW)
    kt = G[:, :, 0:32]
    vt = G[:, :, 32:80]
    gt = G[:, :, 80:83]
    post = G[:, :, 83:86]
    velt = G[:, :, 86:89]
    pos = zb[:, 0:N_POS]
    rel = post - pos[:, None, :]
    dist2 = jnp.sum(rel * rel, axis=-1)
    gamma = gamma_ref[0, 0]
    logits = jnp.sum(q[:, None, :] * kt, axis=-1) / jnp.sqrt(jnp.float32(D_H)) - gamma * dist2
    att = jax.nn.softmax(logits, axis=-1)
    ns = jnp.sum(att[:, :, None] * vt, axis=1)
    new_s = ns @ Wo_ref[...] + bo_ref[...]
    aw = att[:, :, None]
    dpos = jnp.sum(aw * gt[:, :, 0:1] * rel, axis=1)
    dv = jnp.sum(aw * (gt[:, :, 1:2] * rel + gt[:, :, 2:3] * velt), axis=1)
    h = jnp.concatenate([dpos, dv, new_s], axis=-1)
    out_ref[0] = zb + h


def _gather_rows(T, idx2):
    mesh = plsc.VectorSubcoreMesh(core_axis_name="c", subcore_axis_name="s")

    @functools.partial(
        pl.kernel,
        out_type=jax.ShapeDtypeStruct((E, TW), jnp.float32),
        mesh=mesh,
        scratch_types=[
            pltpu.VMEM((EW // 128, 128), jnp.int32),
            pltpu.VMEM((GC, TW), jnp.float32),
            pltpu.SemaphoreType.DMA,
        ],
    )
    def gather_k(T_hbm, idx_hbm, out_hbm, idx_v, rows_v, sem):
        cid = lax.axis_index("c")
        sid = lax.axis_index("s")
        wid = sid * SC_NC + cid
        row_w = pl.multiple_of(wid * (EW // 128), 8)
        pltpu.sync_copy(idx_hbm.at[pl.ds(row_w, EW // 128)], idx_v)

        def body(g, carry):
            base = wid * EW + g * GC
            copies = [
                pltpu.async_copy(
                    T_hbm.at[idx_v.at[g * GS + r]], rows_v.at[pl.ds(r * 128, 128)], sem
                )
                for r in range(GS)
            ]
            for c_ in copies:
                c_.wait()
            pltpu.sync_copy(rows_v, out_hbm.at[pl.ds(base, GC)])
            return carry

        lax.fori_loop(0, NG, body, 0)

    return gather_k(T, idx2)


def kernel(z, t, conditioning, mask, W1, b1, W2, b2, W3, b3, Wq, Wk, Wv, Wo, bo, Wg, bg, gamma):
    del mask
    f32 = jnp.float32

    cond = pl.pallas_call(
        _cond_kernel,
        out_shape=jax.ShapeDtypeStruct((B, D_COND), f32),
    )(t.reshape(B, 1).astype(f32), conditioning,
      W1, b1.reshape(1, -1), W2, b2.reshape(1, -1), W3, b3.reshape(1, -1))
    cond3 = cond.reshape(B, 1, D_COND)

    nbt = N // RB
    q, T = pl.pallas_call(
        _table_kernel,
        grid=(B, nbt),
        in_specs=[
            pl.BlockSpec((1, RB, D_Z), lambda b_, i: (b_, i, 0)),
            pl.BlockSpec((1, 1, D_COND), lambda b_, i: (b_, 0, 0)),
            pl.BlockSpec((D_COND, D_H), lambda b_, i: (0, 0)),
            pl.BlockSpec((D_COND, D_H), lambda b_, i: (0, 0)),
            pl.BlockSpec((D_COND, D_COND), lambda b_, i: (0, 0)),
            pl.BlockSpec((D_COND, 3), lambda b_, i: (0, 0)),
            pl.BlockSpec((1, 3), lambda b_, i: (0, 0)),
        ],
        out_specs=[
            pl.BlockSpec((1, RB, D_H), lambda b_, i: (b_, i, 0)),
            pl.BlockSpec((RB, TW), lambda b_, i: (b_ * nbt + i, 0)),
        ],
        out_shape=[
            jax.ShapeDtypeStruct((B, N, D_H), f32),
            jax.ShapeDtypeStruct((B * N, TW), f32),
        ],
    )(z, cond3, Wq, Wk, Wv, Wg, bg.reshape(1, 3))

    posT = jnp.transpose(z[..., :N_POS], (0, 2, 1))
    tgt = pl.pallas_call(
        _knn_kernel,
        grid=(B, N // RBK),
        in_specs=[
            pl.BlockSpec((1, RBK, D_Z), lambda b_, i: (b_, i, 0)),
            pl.BlockSpec((1, N_POS, N), lambda b_, i: (b_, 0, 0)),
        ],
        out_specs=pl.BlockSpec((1, RBK, K), lambda b_, i: (b_, i, 0)),
        out_shape=jax.ShapeDtypeStruct((B, N, K), jnp.int32),
    )(z, posT)

    gath = _gather_rows(T, tgt.reshape(E // 128, 128))

    nbm = N // RM
    out = pl.pallas_call(
        _mp_kernel,
        grid=(B, nbm),
        in_specs=[
            pl.BlockSpec((1, RM, D_Z), lambda b_, i: (b_, i, 0)),
            pl.BlockSpec((1, RM, D_H), lambda b_, i: (b_, i, 0)),
            pl.BlockSpec((RM * K, TW), lambda b_, i: (b_ * nbm + i, 0)),
            pl.BlockSpec((D_COND, D_COND), lambda b_, i: (0, 0)),
            pl.BlockSpec((1, D_COND), lambda b_, i: (0, 0)),
            pl.BlockSpec((1, 1), lambda b_, i: (0, 0)),
        ],
        out_specs=pl.BlockSpec((1, RM, D_Z), lambda b_, i: (b_, i, 0)),
        out_shape=jax.ShapeDtypeStruct((B, N, D_Z), f32),
    )(z, q, gath, Wo, bo.reshape(1, -1), gamma.reshape(1, 1))

    return out


# speedup vs baseline: 1.0530x; 1.0021x over previous
import functools

import jax
import jax.numpy as jnp
from jax import lax
from jax.experimental import pallas as pl
from jax.experimental.pallas import tpu as pltpu
from jax.experimental.pallas import tpu_sc as plsc

D_T = 32
D_COND = 48
N_POS = 3
K = 20
D_H = 32
B = 2
N = 4096
D_Z = 54

TW = 128
RB = 256
RM = 256
RBK = 256
E = B * N * K

SC_NC = 2
SC_NS = 16
NW = SC_NC * SC_NS
EW = E // NW
GC = 256
GS = GC // 128
NG = EW // GC


def _cond_kernel(t_ref, c_ref, W1_ref, b1_ref, W2_ref, b2_ref, W3_ref, b3_ref, out_ref):
    t = t_ref[...]
    half = D_T // 2
    i = lax.broadcasted_iota(jnp.int32, (1, half), 1).astype(jnp.float32)
    freqs = jnp.exp(-jnp.log(10000.0) * i / (half - 1))
    args = t * freqs
    emb = jnp.concatenate([jnp.sin(args), jnp.cos(args)], axis=-1)
    x = jnp.concatenate([emb, c_ref[...]], axis=-1)
    h = jax.nn.gelu(x @ W1_ref[...] + b1_ref[...])
    h = jax.nn.gelu(h @ W2_ref[...] + b2_ref[...])
    out_ref[...] = h @ W3_ref[...] + b3_ref[...]


def _table_kernel(z_ref, cond_ref, Wq_ref, Wk_ref, Wv_ref, Wg_ref, bg_ref,
                  q_ref, T_ref):
    z = z_ref[0]
    s = z[:, 2 * N_POS:] + cond_ref[0]
    q_ref[0] = s @ Wq_ref[...]
    kt = s @ Wk_ref[...]
    vt = s @ Wv_ref[...]
    gt = s @ Wg_ref[...] + bg_ref[...]
    pos = z[:, 0:N_POS]
    vel = z[:, N_POS:2 * N_POS]
    pad = jnp.zeros((z.shape[0], TW - 89), jnp.float32)
    T_ref[...] = jnp.concatenate([kt, vt, gt, pos, vel, pad], axis=-1)


def _knn_kernel(z_ref, posT_ref, tgt_ref):
    b = pl.program_id(0)
    pr = z_ref[0]
    rows = pr.shape[0]
    d2 = None
    for c in range(N_POS):
        diff = posT_ref[0, c:c + 1, :] - pr[:, c:c + 1]
        d2 = diff * diff if d2 is None else d2 + diff * diff
    iota = lax.broadcasted_iota(jnp.int32, (rows, N), 1)
    bits = lax.bitcast_convert_type(d2, jnp.int32)
    key = lax.bitcast_convert_type(
        ((bits & jnp.int32(-4096)) | iota) + jnp.int32(1 << 23), jnp.float32)
    cols = []
    base = b * N
    for _ in range(K):
        m = jnp.min(key, axis=1, keepdims=True)
        mi = lax.bitcast_convert_type(m, jnp.int32)
        cols.append((mi & jnp.int32(4095)) + base)
        key = jnp.where(key == m, jnp.float32(jnp.inf), key)
    tgt_ref[0] = jnp.concatenate(cols, axis=-1)


def _mp_kernel(z_ref, q_ref, g_ref, Wo_ref, bo_ref, gamma_ref, out_ref):
    zb = z_ref[0]
    q = q_ref[0]
    G = g_ref[...].reshape(RM, K, TW)
    kt = G[:, :, 0:32]
    vt = G[:, :, 32:80]
    gt = G[:, :, 80:83]
    post = G[:, :, 83:86]
    velt = G[:, :, 86:89]
    pos = zb[:, 0:N_POS]
    rel = post - pos[:, None, :]
    dist2 = jnp.sum(rel * rel, axis=-1)
    gamma = gamma_ref[0, 0]
    logits = jnp.sum(q[:, None, :] * kt, axis=-1) / jnp.sqrt(jnp.float32(D_H)) - gamma * dist2
    att = jax.nn.softmax(logits, axis=-1)
    ns = jnp.sum(att[:, :, None] * vt, axis=1)
    new_s = ns @ Wo_ref[...] + bo_ref[...]
    aw = att[:, :, None]
    dpos = jnp.sum(aw * gt[:, :, 0:1] * rel, axis=1)
    dv = jnp.sum(aw * (gt[:, :, 1:2] * rel + gt[:, :, 2:3] * velt), axis=1)
    h = jnp.concatenate([dpos, dv, new_s], axis=-1)
    out_ref[0] = zb + h


def _gather_rows(T, idx2):
    mesh = plsc.VectorSubcoreMesh(core_axis_name="c", subcore_axis_name="s")

    @functools.partial(
        pl.kernel,
        out_type=jax.ShapeDtypeStruct((E, TW), jnp.float32),
        mesh=mesh,
        scratch_types=[
            pltpu.VMEM((EW // 128, 128), jnp.int32),
            pltpu.VMEM((GC, TW), jnp.float32),
            pltpu.VMEM((GC, TW), jnp.float32),
            pltpu.SemaphoreType.DMA,
            pltpu.SemaphoreType.DMA,
        ],
    )
    def gather_k(T_hbm, idx_hbm, out_hbm, idx_v, rows_a, rows_b, sem_a, sem_b):
        cid = lax.axis_index("c")
        sid = lax.axis_index("s")
        wid = sid * SC_NC + cid
        row_w = pl.multiple_of(wid * (EW // 128), 8)
        pltpu.sync_copy(idx_hbm.at[pl.ds(row_w, EW // 128)], idx_v)

        def fire(g, rows_v, sem):
            return [
                pltpu.async_copy(
                    T_hbm.at[idx_v.at[g * GS + r]], rows_v.at[pl.ds(r * 128, 128)], sem
                )
                for r in range(GS)
            ]

        def drain(copies):
            for c_ in copies:
                c_.wait()

        fire(0, rows_a, sem_a)

        def body(h, carry):
            g0 = 2 * h
            g1 = 2 * h + 1
            g2 = jnp.where(g1 + 1 < NG, g1 + 1, 0)
            drain([pltpu.make_async_copy(
                T_hbm.at[idx_v.at[g0 * GS + r]], rows_a.at[pl.ds(r * 128, 128)], sem_a)
                for r in range(GS)])
            fire(g1, rows_b, sem_b)
            pltpu.sync_copy(rows_a, out_hbm.at[pl.ds(wid * EW + g0 * GC, GC)])
            drain([pltpu.make_async_copy(
                T_hbm.at[idx_v.at[g1 * GS + r]], rows_b.at[pl.ds(r * 128, 128)], sem_b)
                for r in range(GS)])
            fire(g2, rows_a, sem_a)
            pltpu.sync_copy(rows_b, out_hbm.at[pl.ds(wid * EW + g1 * GC, GC)])
            return carry

        lax.fori_loop(0, NG // 2, body, 0)
        drain([pltpu.make_async_copy(
            T_hbm.at[idx_v.at[r]], rows_a.at[pl.ds(r * 128, 128)], sem_a)
            for r in range(GS)])

    return gather_k(T, idx2)


def kernel(z, t, conditioning, mask, W1, b1, W2, b2, W3, b3, Wq, Wk, Wv, Wo, bo, Wg, bg, gamma):
    del mask
    f32 = jnp.float32

    cond = pl.pallas_call(
        _cond_kernel,
        out_shape=jax.ShapeDtypeStruct((B, D_COND), f32),
    )(t.reshape(B, 1).astype(f32), conditioning,
      W1, b1.reshape(1, -1), W2, b2.reshape(1, -1), W3, b3.reshape(1, -1))
    cond3 = cond.reshape(B, 1, D_COND)

    nbt = N // RB
    q, T = pl.pallas_call(
        _table_kernel,
        grid=(B, nbt),
        in_specs=[
            pl.BlockSpec((1, RB, D_Z), lambda b_, i: (b_, i, 0)),
            pl.BlockSpec((1, 1, D_COND), lambda b_, i: (b_, 0, 0)),
            pl.BlockSpec((D_COND, D_H), lambda b_, i: (0, 0)),
            pl.BlockSpec((D_COND, D_H), lambda b_, i: (0, 0)),
            pl.BlockSpec((D_COND, D_COND), lambda b_, i: (0, 0)),
            pl.BlockSpec((D_COND, 3), lambda b_, i: (0, 0)),
            pl.BlockSpec((1, 3), lambda b_, i: (0, 0)),
        ],
        out_specs=[
            pl.BlockSpec((1, RB, D_H), lambda b_, i: (b_, i, 0)),
            pl.BlockSpec((RB, TW), lambda b_, i: (b_ * nbt + i, 0)),
        ],
        out_shape=[
            jax.ShapeDtypeStruct((B, N, D_H), f32),
            jax.ShapeDtypeStruct((B * N, TW), f32),
        ],
    )(z, cond3, Wq, Wk, Wv, Wg, bg.reshape(1, 3))

    posT = jnp.transpose(z[..., :N_POS], (0, 2, 1))
    tgt = pl.pallas_call(
        _knn_kernel,
        grid=(B, N // RBK),
        in_specs=[
            pl.BlockSpec((1, RBK, D_Z), lambda b_, i: (b_, i, 0)),
            pl.BlockSpec((1, N_POS, N), lambda b_, i: (b_, 0, 0)),
        ],
        out_specs=pl.BlockSpec((1, RBK, K), lambda b_, i: (b_, i, 0)),
        out_shape=jax.ShapeDtypeStruct((B, N, K), jnp.int32),
    )(z, posT)

    gath = _gather_rows(T, tgt.reshape(E // 128, 128))

    nbm = N // RM
    out = pl.pallas_call(
        _mp_kernel,
        grid=(B, nbm),
        in_specs=[
            pl.BlockSpec((1, RM, D_Z), lambda b_, i: (b_, i, 0)),
            pl.BlockSpec((1, RM, D_H), lambda b_, i: (b_, i, 0)),
            pl.BlockSpec((RM * K, TW), lambda b_, i: (b_ * nbm + i, 0)),
            pl.BlockSpec((D_COND, D_COND), lambda b_, i: (0, 0)),
            pl.BlockSpec((1, D_COND), lambda b_, i: (0, 0)),
            pl.BlockSpec((1, 1), lambda b_, i: (0, 0)),
        ],
        out_specs=pl.BlockSpec((1, RM, D_Z), lambda b_, i: (b_, i, 0)),
        out_shape=jax.ShapeDtypeStruct((B, N, D_Z), f32),
    )(z, q, gath, Wo, bo.reshape(1, -1), gamma.reshape(1, 1))

    return out
